# baseline (speedup 1.0000x reference)
import jax
import jax.numpy as jnp
from jax.experimental import pallas as pl
from jax.experimental.pallas import tpu as pltpu

_TOP_K = 8
_E = 64
_ALPHA = 0.001
_H = 1024
_BSZ = 4
_SEQ = 8192
_N_TOK = _BSZ * _SEQ
_BLOCK = 2048
_GRID = _N_TOK // _BLOCK
_BLOCKS_PER_BATCH = _SEQ // _BLOCK
_COEF = _ALPHA * _E / (_BSZ * _SEQ * _SEQ * _TOP_K)


def _gate_block(x_ref, w_ref, idx_ref, wgt_ref, aux_ref, acc_ref):
    pid = pl.program_id(0)
    b = pid // _BLOCKS_PER_BATCH

    @pl.when(pid == 0)
    def _init():
        acc_ref[...] = jnp.zeros_like(acc_ref)

    logits = jax.lax.dot_general(
        w_ref[...], x_ref[...],
        dimension_numbers=(((1,), (1,)), ((), ())),
        preferred_element_type=jnp.float32,
        precision=jax.lax.Precision.DEFAULT,
    )
    m = jnp.max(logits, axis=0, keepdims=True)
    ex = jnp.exp(logits - m)
    probs = ex / jnp.sum(ex, axis=0, keepdims=True)

    iota = jax.lax.broadcasted_iota(jnp.int32, (_E, _BLOCK), 0)
    s = probs
    vals = []
    idxs = []
    for _ in range(_TOP_K):
        mv = jnp.max(s, axis=0, keepdims=True)
        idx = jnp.min(jnp.where(s == mv, iota, _E), axis=0, keepdims=True)
        vals.append(mv)
        idxs.append(idx)
        s = jnp.where(iota == idx, -1.0, s)

    topv = jnp.concatenate(vals, axis=0)
    denom = jnp.sum(topv, axis=0, keepdims=True) + 1e-20
    wgt_ref[...] = topv / denom
    idx_ref[...] = jnp.concatenate(idxs, axis=0)

    cnt = jnp.sum(jnp.where(s < 0.0, 1.0, 0.0), axis=1, keepdims=True)
    ssum = jnp.sum(probs, axis=1, keepdims=True)
    c_iota = jax.lax.broadcasted_iota(jnp.int32, (_E, 2 * _BSZ), 1)
    acc_ref[...] = (acc_ref[...]
                    + jnp.where(c_iota == b, cnt, 0.0)
                    + jnp.where(c_iota == b + _BSZ, ssum, 0.0))

    @pl.when(pid == _GRID - 1)
    def _finish():
        acc = acc_ref[...]
        aux = _COEF * jnp.sum(acc[:, :_BSZ] * acc[:, _BSZ:])
        aux_ref[...] = aux * jnp.ones((1, 1), jnp.float32)


def _moe_gate(x_flat, weight, interpret=False):
    idx_t, wgt_t, aux = pl.pallas_call(
        _gate_block,
        grid=(_GRID,),
        in_specs=[
            pl.BlockSpec((_BLOCK, _H), lambda i: (i, 0)),
            pl.BlockSpec((_E, _H), lambda i: (0, 0)),
        ],
        out_specs=[
            pl.BlockSpec((_TOP_K, _BLOCK), lambda i: (0, i)),
            pl.BlockSpec((_TOP_K, _BLOCK), lambda i: (0, i)),
            pl.BlockSpec((1, 1), lambda i: (0, 0)),
        ],
        out_shape=[
            jax.ShapeDtypeStruct((_TOP_K, _N_TOK), jnp.int32),
            jax.ShapeDtypeStruct((_TOP_K, _N_TOK), jnp.float32),
            jax.ShapeDtypeStruct((1, 1), jnp.float32),
        ],
        scratch_shapes=[pltpu.VMEM((_E, 2 * _BSZ), jnp.float32)],
        interpret=interpret,
    )(x_flat, weight)
    return idx_t, wgt_t, aux


def kernel(hidden_states, weight):
    bsz, seq_len, h = hidden_states.shape
    x_flat = hidden_states.reshape(-1, h)
    idx_t, wgt_t, aux = _moe_gate(x_flat, weight)
    return idx_t, wgt_t, aux[0, 0]


# speedup vs baseline: 1.0650x; 1.0650x over previous
import jax
import jax.numpy as jnp
from jax.experimental import pallas as pl
from jax.experimental.pallas import tpu as pltpu

_TOP_K = 8
_E = 64
_ALPHA = 0.001
_H = 1024
_BSZ = 4
_SEQ = 8192
_N_TOK = _BSZ * _SEQ
_BLOCK = 4096
_GRID = _N_TOK // _BLOCK
_BLOCKS_PER_BATCH = _SEQ // _BLOCK
_COEF = _ALPHA * _E / (_BSZ * _SEQ * _SEQ * _TOP_K)


def _gate_block(x_ref, w_ref, idx_ref, wgt_ref, aux_ref, acc_ref):
    pid = pl.program_id(0)
    b = pid // _BLOCKS_PER_BATCH

    @pl.when(pid == 0)
    def _init():
        acc_ref[...] = jnp.zeros_like(acc_ref)

    logits = jax.lax.dot_general(
        w_ref[...], x_ref[...],
        dimension_numbers=(((1,), (1,)), ((), ())),
        preferred_element_type=jnp.float32,
        precision=jax.lax.Precision.DEFAULT,
    )
    m = jnp.max(logits, axis=0, keepdims=True)
    ex = jnp.exp(logits - m)
    probs = ex / jnp.sum(ex, axis=0, keepdims=True)

    iota = jax.lax.broadcasted_iota(jnp.int32, (_E, _BLOCK), 0)
    s = probs
    vals = []
    idxs = []
    for _ in range(_TOP_K):
        mv = jnp.max(s, axis=0, keepdims=True)
        idx = jnp.min(jnp.where(s == mv, iota, _E), axis=0, keepdims=True)
        vals.append(mv)
        idxs.append(idx)
        s = jnp.where(iota == idx, -1.0, s)

    topv = jnp.concatenate(vals, axis=0)
    denom = jnp.sum(topv, axis=0, keepdims=True) + 1e-20
    wgt_ref[...] = topv / denom
    idx_ref[...] = jnp.concatenate(idxs, axis=0)

    cnt = jnp.sum(jnp.where(s < 0.0, 1.0, 0.0), axis=1, keepdims=True)
    ssum = jnp.sum(probs, axis=1, keepdims=True)
    c_iota = jax.lax.broadcasted_iota(jnp.int32, (_E, 2 * _BSZ), 1)
    acc_ref[...] = (acc_ref[...]
                    + jnp.where(c_iota == b, cnt, 0.0)
                    + jnp.where(c_iota == b + _BSZ, ssum, 0.0))

    @pl.when(pid == _GRID - 1)
    def _finish():
        acc = acc_ref[...]
        aux = _COEF * jnp.sum(acc[:, :_BSZ] * acc[:, _BSZ:])
        aux_ref[...] = aux * jnp.ones((1, 1), jnp.float32)


def _moe_gate(x_flat, weight, interpret=False):
    idx_t, wgt_t, aux = pl.pallas_call(
        _gate_block,
        grid=(_GRID,),
        in_specs=[
            pl.BlockSpec((_BLOCK, _H), lambda i: (i, 0)),
            pl.BlockSpec((_E, _H), lambda i: (0, 0)),
        ],
        out_specs=[
            pl.BlockSpec((_TOP_K, _BLOCK), lambda i: (0, i)),
            pl.BlockSpec((_TOP_K, _BLOCK), lambda i: (0, i)),
            pl.BlockSpec((1, 1), lambda i: (0, 0)),
        ],
        out_shape=[
            jax.ShapeDtypeStruct((_TOP_K, _N_TOK), jnp.int32),
            jax.ShapeDtypeStruct((_TOP_K, _N_TOK), jnp.float32),
            jax.ShapeDtypeStruct((1, 1), jnp.float32),
        ],
        scratch_shapes=[pltpu.VMEM((_E, 2 * _BSZ), jnp.float32)],
        interpret=interpret,
    )(x_flat, weight)
    return idx_t, wgt_t, aux


def kernel(hidden_states, weight):
    bsz, seq_len, h = hidden_states.shape
    x_flat = hidden_states.reshape(-1, h)
    idx_t, wgt_t, aux = _moe_gate(x_flat, weight)
    return idx_t.T, wgt_t.T, aux[0, 0]


# speedup vs baseline: 1.1123x; 1.0444x over previous
import jax
import jax.numpy as jnp
from jax.experimental import pallas as pl
from jax.experimental.pallas import tpu as pltpu

_TOP_K = 8
_E = 64
_ALPHA = 0.001
_H = 1024
_BSZ = 4
_SEQ = 8192
_N_TOK = _BSZ * _SEQ
_BLOCK = 4096
_GRID = _N_TOK // _BLOCK
_BLOCKS_PER_BATCH = _SEQ // _BLOCK
_COEF = _ALPHA * _E / (_BSZ * _SEQ * _SEQ * _TOP_K)


def _gate_block(x_ref, w_ref, idx_ref, wgt_ref, aux_ref, acc_ref):
    pid = pl.program_id(0)
    b = pid // _BLOCKS_PER_BATCH

    @pl.when(pid == 0)
    def _init():
        acc_ref[...] = jnp.zeros_like(acc_ref)

    logits = jax.lax.dot_general(
        w_ref[...], x_ref[...],
        dimension_numbers=(((1,), (1,)), ((), ())),
        preferred_element_type=jnp.float32,
        precision=jax.lax.Precision.DEFAULT,
    )
    m = jnp.max(logits, axis=0, keepdims=True)
    ex = jnp.exp(logits - m)
    probs = ex / jnp.sum(ex, axis=0, keepdims=True)

    iota_f = jax.lax.broadcasted_iota(
        jnp.int32, (_E, _BLOCK), 0).astype(jnp.float32)
    s = probs
    vals = []
    idxs = []
    for _ in range(_TOP_K):
        mv = jnp.max(s, axis=0, keepdims=True)
        idx = jnp.min(jnp.where(s == mv, iota_f, float(_E)),
                      axis=0, keepdims=True)
        vals.append(mv)
        idxs.append(idx)
        s = jnp.where(iota_f == idx, -1.0, s)

    topv = jnp.concatenate(vals, axis=0)
    denom = jnp.sum(topv, axis=0, keepdims=True) + 1e-20
    wgt_ref[...] = topv / denom
    idx_ref[...] = jnp.concatenate(idxs, axis=0).astype(jnp.int32)

    cnt = jnp.sum(jnp.where(s < 0.0, 1.0, 0.0), axis=1, keepdims=True)
    ssum = jnp.sum(probs, axis=1, keepdims=True)
    c_iota = jax.lax.broadcasted_iota(jnp.int32, (_E, 2 * _BSZ), 1)
    acc_ref[...] = (acc_ref[...]
                    + jnp.where(c_iota == b, cnt, 0.0)
                    + jnp.where(c_iota == b + _BSZ, ssum, 0.0))

    @pl.when(pid == _GRID - 1)
    def _finish():
        acc = acc_ref[...]
        aux = _COEF * jnp.sum(acc[:, :_BSZ] * acc[:, _BSZ:])
        aux_ref[...] = aux * jnp.ones((1, 1), jnp.float32)


def _moe_gate(x_flat, weight, interpret=False):
    idx_t, wgt_t, aux = pl.pallas_call(
        _gate_block,
        grid=(_GRID,),
        in_specs=[
            pl.BlockSpec((_BLOCK, _H), lambda i: (i, 0)),
            pl.BlockSpec((_E, _H), lambda i: (0, 0)),
        ],
        out_specs=[
            pl.BlockSpec((_TOP_K, _BLOCK), lambda i: (0, i)),
            pl.BlockSpec((_TOP_K, _BLOCK), lambda i: (0, i)),
            pl.BlockSpec((1, 1), lambda i: (0, 0)),
        ],
        out_shape=[
            jax.ShapeDtypeStruct((_TOP_K, _N_TOK), jnp.int32),
            jax.ShapeDtypeStruct((_TOP_K, _N_TOK), jnp.float32),
            jax.ShapeDtypeStruct((1, 1), jnp.float32),
        ],
        scratch_shapes=[pltpu.VMEM((_E, 2 * _BSZ), jnp.float32)],
        interpret=interpret,
    )(x_flat, weight)
    return idx_t, wgt_t, aux


def kernel(hidden_states, weight):
    bsz, seq_len, h = hidden_states.shape
    x_flat = hidden_states.reshape(-1, h)
    idx_t, wgt_t, aux = _moe_gate(x_flat, weight)
    return idx_t.T, wgt_t.T, aux[0, 0]
